# Initial kernel scaffold; baseline (speedup 1.0000x reference)
import jax
import jax.numpy as jnp
from jax.experimental import pallas as pl


def kernel(x, edge_index, edge_type, gene_emb, w1, root1, b1, w2, root2, b2, lw1, lb1, lw2, lb2):
    raise NotImplementedError("write your pallas kernel here")


# speedup vs baseline: 3.3230x; 3.3230x over previous
import functools

import jax
import jax.numpy as jnp
from jax import lax
from jax.experimental import pallas as pl
from jax.experimental.pallas import tpu as pltpu
from jax.experimental.pallas import tpu_sc as plsc

N_X = 5736
N_GENE = 4264
N = N_X + N_GENE
E = 160000
NREL = 5

B_EDGE = 512
R_NODE = 200
NB = N // R_NODE
E_PAD = ((E + B_EDGE - 1) // B_EDGE) * B_EDGE
NCH = E_PAD // B_EDGE


def _mm_body(a_ref, b_ref, o_ref):
    o_ref[...] = jnp.dot(a_ref[...], b_ref[...],
                         preferred_element_type=jnp.float32)


def _matmul(a, b, bm=200, bn=256):
    m, k = a.shape
    k2, n = b.shape
    assert k == k2 and m % bm == 0 and n % bn == 0
    return pl.pallas_call(
        _mm_body,
        grid=(m // bm, n // bn),
        in_specs=[
            pl.BlockSpec((bm, k), lambda i, j: (i, 0)),
            pl.BlockSpec((k, bn), lambda i, j: (0, j)),
        ],
        out_specs=pl.BlockSpec((bm, bn), lambda i, j: (i, j)),
        out_shape=jax.ShapeDtypeStruct((m, n), jnp.float32),
    )(a, b)


def _sc_gather(table, idx, d):
    info = plsc.get_sparse_core_info()
    nc, ns = info.num_cores, info.num_subcores
    nw = nc * ns
    b_per_w = E_PAD // nw
    chunk = 8
    nsteps = b_per_w // chunk
    mesh = plsc.VectorSubcoreMesh(core_axis_name="c", subcore_axis_name="s")

    @functools.partial(
        pl.kernel, mesh=mesh,
        out_type=jax.ShapeDtypeStruct((E_PAD, d), jnp.float32),
        scratch_types=[
            pltpu.VMEM((chunk,), jnp.int32),
            pltpu.VMEM((chunk, d), jnp.float32),
            pltpu.SemaphoreType.DMA,
        ],
    )
    def gather_k(table_hbm, idx_hbm, out_hbm, idx_v, rows_v, sem):
        wid = lax.axis_index("s") * nc + lax.axis_index("c")
        base = wid * b_per_w

        def body(i, carry):
            off = base + i * chunk
            pltpu.sync_copy(idx_hbm.at[pl.ds(off, chunk)], idx_v)
            pltpu.async_copy(table_hbm.at[idx_v], rows_v, sem).wait()
            pltpu.sync_copy(rows_v, out_hbm.at[pl.ds(off, chunk)])
            return carry

        lax.fori_loop(0, nsteps, body, 0)

    return gather_k(table, idx)


def _seg_body(offs_ref, msg_hbm, code_hbm, out0_ref, out_ref,
              code_v, msg_v, sem, *, dp):
    k = pl.program_id(0)
    lo = offs_ref[k]
    hi = offs_ref[k + 1]
    c_lo = lo // B_EDGE
    c_hi = (hi + B_EDGE - 1) // B_EDGE

    iota_col = lax.broadcasted_iota(jnp.int32, (R_NODE, B_EDGE), 0)

    def body(c, carry):
        s, cnt = carry
        pltpu.make_async_copy(code_hbm.at[c], code_v, sem).start()
        pltpu.make_async_copy(code_hbm.at[c], code_v, sem).wait()
        pltpu.make_async_copy(
            msg_hbm.at[(pl.ds(c * B_EDGE, B_EDGE), slice(None))], msg_v,
            sem).start()
        pltpu.make_async_copy(
            msg_hbm.at[(pl.ds(c * B_EDGE, B_EDGE), slice(None))], msg_v,
            sem).wait()
        code = code_v[...]
        dloc = code // 8 - k * R_NODE
        rel = code % 8
        dloc_b = jnp.broadcast_to(dloc, (R_NODE, B_EDGE))
        rel_b = jnp.broadcast_to(rel, (R_NODE, B_EDGE))
        msg = msg_v[...]
        s_new = []
        cnt_new = []
        for r in range(NREL):
            oht = jnp.where((dloc_b == iota_col) & (rel_b == r), 1.0, 0.0)
            s_new.append(s[r] + jnp.dot(oht, msg,
                                        preferred_element_type=jnp.float32))
            cnt_new.append(cnt[r] + jnp.sum(oht, axis=1, keepdims=True))
        return tuple(s_new), tuple(cnt_new)

    zero_s = tuple(jnp.zeros((R_NODE, dp), jnp.float32) for _ in range(NREL))
    zero_c = tuple(jnp.zeros((R_NODE, 1), jnp.float32) for _ in range(NREL))
    s, cnt = lax.fori_loop(c_lo, c_hi, body, (zero_s, zero_c))

    out = out0_ref[...]
    for r in range(NREL):
        out = out + s[r] / jnp.maximum(cnt[r], 1.0)
    out_ref[...] = jnp.maximum(out, 0.0)


def _seg_aggregate(offs, msg, code3d, out0, dp):
    grid_spec = pltpu.PrefetchScalarGridSpec(
        num_scalar_prefetch=1,
        grid=(NB,),
        in_specs=[
            pl.BlockSpec(memory_space=pltpu.MemorySpace.HBM),
            pl.BlockSpec(memory_space=pltpu.MemorySpace.HBM),
            pl.BlockSpec((R_NODE, dp), lambda k, offs: (k, 0)),
        ],
        out_specs=pl.BlockSpec((R_NODE, dp), lambda k, offs: (k, 0)),
        scratch_shapes=[
            pltpu.VMEM((1, B_EDGE), jnp.int32),
            pltpu.VMEM((B_EDGE, dp), jnp.float32),
            pltpu.SemaphoreType.DMA,
        ],
    )
    return pl.pallas_call(
        functools.partial(_seg_body, dp=dp),
        grid_spec=grid_spec,
        out_shape=jax.ShapeDtypeStruct((N, dp), jnp.float32),
    )(offs, msg, code3d, out0)


def _head_body(h_ref, lw1_ref, lb1_ref, lw2_ref, lb2_ref, logp_ref, emb_ref):
    z1 = jnp.maximum(
        jnp.dot(h_ref[...], lw1_ref[...], preferred_element_type=jnp.float32)
        + lb1_ref[...], 0.0)
    logits = jnp.dot(z1, lw2_ref[...],
                     preferred_element_type=jnp.float32) + lb2_ref[...]
    l0 = logits[:, 0:1]
    l1 = logits[:, 1:2]
    m = jnp.maximum(l0, l1)
    lse = m + jnp.log(jnp.exp(l0 - m) + jnp.exp(l1 - m))
    logp_ref[...] = logits - lse
    emb_ref[...] = z1


def _head(h, lw1p, lb1p, lw2p, lb2p, dp, d3p):
    return pl.pallas_call(
        _head_body,
        grid=(NB,),
        in_specs=[
            pl.BlockSpec((R_NODE, dp), lambda k: (k, 0)),
            pl.BlockSpec((dp, d3p), lambda k: (0, 0)),
            pl.BlockSpec((1, d3p), lambda k: (0, 0)),
            pl.BlockSpec((d3p, 128), lambda k: (0, 0)),
            pl.BlockSpec((1, 128), lambda k: (0, 0)),
        ],
        out_specs=(
            pl.BlockSpec((R_NODE, 128), lambda k: (k, 0)),
            pl.BlockSpec((R_NODE, d3p), lambda k: (k, 0)),
        ),
        out_shape=(
            jax.ShapeDtypeStruct((N, 128), jnp.float32),
            jax.ShapeDtypeStruct((N, d3p), jnp.float32),
        ),
    )(h, lw1p, lb1p, lw2p, lb2p)


def _pad2(a, rows, cols):
    return jnp.pad(a, ((0, rows - a.shape[0]), (0, cols - a.shape[1])))


def _rgcn_layer(h, w, root, b, gidx, code3d, offs, din_p, dout_p):
    blocks = [_pad2(w[r], din_p, dout_p) for r in range(NREL)]
    blocks.append(_pad2(root, din_p, dout_p))
    wcat = jnp.concatenate(blocks, axis=1)
    t = _matmul(h, wcat)
    table = t[:, :NREL * dout_p].reshape(N * NREL, dout_p)
    bp = jnp.pad(b, (0, dout_p - b.shape[0]))
    out0 = t[:, NREL * dout_p:] + bp[None, :]
    msg = _sc_gather(table, gidx, dout_p)
    return _seg_aggregate(offs, msg, code3d, out0, dout_p)


def kernel(x, edge_index, edge_type, gene_emb, w1, root1, b1, w2, root2, b2,
           lw1, lb1, lw2, lb2):
    h = jnp.concatenate([x, gene_emb], axis=0)
    h = jnp.pad(h, ((0, 0), (0, 3)))

    src = edge_index[0]
    dst = edge_index[1]
    perm = jnp.argsort(dst)
    sdst = dst[perm]
    ssrc = src[perm]
    srel = edge_type[perm]

    gidx = jnp.pad(ssrc * NREL + srel, (0, E_PAD - E)).astype(jnp.int32)
    code = jnp.pad(sdst * 8 + srel, (0, E_PAD - E),
                   constant_values=8 * N).astype(jnp.int32)
    code3d = code.reshape(NCH, 1, B_EDGE)
    offs = jnp.searchsorted(
        sdst, jnp.arange(NB + 1, dtype=jnp.int32) * R_NODE).astype(jnp.int32)

    d1p, d2p, d3p = 1408, 1024, 768
    w1l = [jnp.pad(w1[r], ((0, 3), (0, 0))) for r in range(NREL)]
    w1p = jnp.stack(w1l)
    root1p = jnp.pad(root1, ((0, 3), (0, 0)))
    h1 = _rgcn_layer(h, w1p, root1p, b1, gidx, code3d, offs, 1616, d1p)
    h2 = _rgcn_layer(h1, w2, root2, b2, gidx, code3d, offs, d1p, d2p)

    lw1p = _pad2(lw1, d2p, d3p)
    lb1p = jnp.pad(lb1, (0, d3p - lb1.shape[0]))[None, :]
    lw2p = _pad2(lw2, d3p, 128)
    lb2p = jnp.pad(lb2, (0, 128 - lb2.shape[0]))[None, :]
    logp_pad, emb_pad = _head(h2, lw1p, lb1p, lw2p, lb2p, d2p, d3p)
    return (logp_pad[:, :2], emb_pad[:, :740])


# speedup vs baseline: 3.5722x; 1.0750x over previous
import functools

import jax
import jax.numpy as jnp
from jax import lax
from jax.experimental import pallas as pl
from jax.experimental.pallas import tpu as pltpu
from jax.experimental.pallas import tpu_sc as plsc

N_X = 5736
N_GENE = 4264
N = N_X + N_GENE
E = 160000
NREL = 5

B_EDGE = 512
R_NODE = 200
NB = N // R_NODE
E_PAD = ((E + B_EDGE - 1) // B_EDGE) * B_EDGE
NCH = E_PAD // B_EDGE


def _mm_body(a_ref, b_ref, o_ref):
    o_ref[...] = jnp.dot(a_ref[...], b_ref[...],
                         preferred_element_type=jnp.float32)


def _matmul(a, b, bm=200, bn=256):
    m, k = a.shape
    k2, n = b.shape
    assert k == k2 and m % bm == 0 and n % bn == 0
    return pl.pallas_call(
        _mm_body,
        grid=(m // bm, n // bn),
        in_specs=[
            pl.BlockSpec((bm, k), lambda i, j: (i, 0)),
            pl.BlockSpec((k, bn), lambda i, j: (0, j)),
        ],
        out_specs=pl.BlockSpec((bm, bn), lambda i, j: (i, j)),
        out_shape=jax.ShapeDtypeStruct((m, n), jnp.float32),
    )(a, b)


def _sc_gather(table, idx, d):
    info = plsc.get_sparse_core_info()
    nc, ns = info.num_cores, info.num_subcores
    nw = nc * ns
    b_per_w = E_PAD // nw
    chunk = 16
    nsteps = b_per_w // chunk
    mesh = plsc.VectorSubcoreMesh(core_axis_name="c", subcore_axis_name="s")

    @functools.partial(
        pl.kernel, mesh=mesh,
        out_type=jax.ShapeDtypeStruct((E_PAD, d), jnp.float32),
        scratch_types=[
            pltpu.VMEM((chunk,), jnp.int32),
            pltpu.VMEM((chunk, d), jnp.float32),
            pltpu.SemaphoreType.DMA,
        ],
    )
    def gather_k(table_hbm, idx_hbm, out_hbm, idx_v, rows_v, sem):
        wid = lax.axis_index("s") * nc + lax.axis_index("c")
        base = wid * b_per_w

        def body(i, carry):
            off = base + i * chunk
            pltpu.sync_copy(idx_hbm.at[pl.ds(off, chunk)], idx_v)
            pltpu.async_copy(table_hbm.at[idx_v], rows_v, sem).wait()
            pltpu.sync_copy(rows_v, out_hbm.at[pl.ds(off, chunk)])
            return carry

        lax.fori_loop(0, nsteps, body, 0)

    return gather_k(table, idx)


def _seg_body(offs_ref, msg_hbm, code_hbm, out0_ref, out_ref,
              code_v, msg_v, sem, *, dp):
    k = pl.program_id(0)
    lo = offs_ref[k]
    hi = offs_ref[k + 1]
    c_lo = lo // B_EDGE
    c_hi = (hi + B_EDGE - 1) // B_EDGE

    iota_col = lax.broadcasted_iota(jnp.int32, (R_NODE, B_EDGE), 0)

    def body(c, carry):
        s, cnt = carry
        pltpu.make_async_copy(code_hbm.at[c], code_v, sem).start()
        pltpu.make_async_copy(code_hbm.at[c], code_v, sem).wait()
        pltpu.make_async_copy(
            msg_hbm.at[(pl.ds(c * B_EDGE, B_EDGE), slice(None))], msg_v,
            sem).start()
        pltpu.make_async_copy(
            msg_hbm.at[(pl.ds(c * B_EDGE, B_EDGE), slice(None))], msg_v,
            sem).wait()
        code = code_v[...]
        dloc = code // 8 - k * R_NODE
        rel = code % 8
        dloc_b = jnp.broadcast_to(dloc, (R_NODE, B_EDGE))
        rel_b = jnp.broadcast_to(rel, (R_NODE, B_EDGE))
        msg = msg_v[...]
        s_new = []
        cnt_new = []
        for r in range(NREL):
            oht = jnp.where((dloc_b == iota_col) & (rel_b == r), 1.0, 0.0)
            s_new.append(s[r] + jnp.dot(oht, msg,
                                        preferred_element_type=jnp.float32))
            cnt_new.append(cnt[r] + jnp.sum(oht, axis=1, keepdims=True))
        return tuple(s_new), tuple(cnt_new)

    zero_s = tuple(jnp.zeros((R_NODE, dp), jnp.float32) for _ in range(NREL))
    zero_c = tuple(jnp.zeros((R_NODE, 1), jnp.float32) for _ in range(NREL))
    s, cnt = lax.fori_loop(c_lo, c_hi, body, (zero_s, zero_c))

    out = out0_ref[...]
    for r in range(NREL):
        out = out + s[r] / jnp.maximum(cnt[r], 1.0)
    out_ref[...] = jnp.maximum(out, 0.0)


def _seg_aggregate(offs, msg, code3d, out0, dp):
    grid_spec = pltpu.PrefetchScalarGridSpec(
        num_scalar_prefetch=1,
        grid=(NB,),
        in_specs=[
            pl.BlockSpec(memory_space=pltpu.MemorySpace.HBM),
            pl.BlockSpec(memory_space=pltpu.MemorySpace.HBM),
            pl.BlockSpec((R_NODE, dp), lambda k, offs: (k, 0)),
        ],
        out_specs=pl.BlockSpec((R_NODE, dp), lambda k, offs: (k, 0)),
        scratch_shapes=[
            pltpu.VMEM((1, B_EDGE), jnp.int32),
            pltpu.VMEM((B_EDGE, dp), jnp.float32),
            pltpu.SemaphoreType.DMA,
        ],
    )
    return pl.pallas_call(
        functools.partial(_seg_body, dp=dp),
        grid_spec=grid_spec,
        out_shape=jax.ShapeDtypeStruct((N, dp), jnp.float32),
    )(offs, msg, code3d, out0)


def _head_body(h_ref, lw1_ref, lb1_ref, lw2_ref, lb2_ref, logp_ref, emb_ref):
    z1 = jnp.maximum(
        jnp.dot(h_ref[...], lw1_ref[...], preferred_element_type=jnp.float32)
        + lb1_ref[...], 0.0)
    logits = jnp.dot(z1, lw2_ref[...],
                     preferred_element_type=jnp.float32) + lb2_ref[...]
    l0 = logits[:, 0:1]
    l1 = logits[:, 1:2]
    m = jnp.maximum(l0, l1)
    lse = m + jnp.log(jnp.exp(l0 - m) + jnp.exp(l1 - m))
    logp_ref[...] = logits - lse
    emb_ref[...] = z1


def _head(h, lw1p, lb1p, lw2p, lb2p, dp, d3p):
    return pl.pallas_call(
        _head_body,
        grid=(NB,),
        in_specs=[
            pl.BlockSpec((R_NODE, dp), lambda k: (k, 0)),
            pl.BlockSpec((dp, d3p), lambda k: (0, 0)),
            pl.BlockSpec((1, d3p), lambda k: (0, 0)),
            pl.BlockSpec((d3p, 128), lambda k: (0, 0)),
            pl.BlockSpec((1, 128), lambda k: (0, 0)),
        ],
        out_specs=(
            pl.BlockSpec((R_NODE, 128), lambda k: (k, 0)),
            pl.BlockSpec((R_NODE, d3p), lambda k: (k, 0)),
        ),
        out_shape=(
            jax.ShapeDtypeStruct((N, 128), jnp.float32),
            jax.ShapeDtypeStruct((N, d3p), jnp.float32),
        ),
    )(h, lw1p, lb1p, lw2p, lb2p)


def _pad2(a, rows, cols):
    return jnp.pad(a, ((0, rows - a.shape[0]), (0, cols - a.shape[1])))


def _rgcn_layer(h, w, root, b, gidx, code3d, offs, din_p, dout_p):
    blocks = [_pad2(w[r], din_p, dout_p) for r in range(NREL)]
    blocks.append(_pad2(root, din_p, dout_p))
    wcat = jnp.concatenate(blocks, axis=1)
    t = _matmul(h, wcat)
    table = t[:, :NREL * dout_p].reshape(N * NREL, dout_p)
    bp = jnp.pad(b, (0, dout_p - b.shape[0]))
    out0 = t[:, NREL * dout_p:] + bp[None, :]
    msg = _sc_gather(table, gidx, dout_p)
    return _seg_aggregate(offs, msg, code3d, out0, dout_p)


def kernel(x, edge_index, edge_type, gene_emb, w1, root1, b1, w2, root2, b2,
           lw1, lb1, lw2, lb2):
    h = jnp.concatenate([x, gene_emb], axis=0)
    h = jnp.pad(h, ((0, 0), (0, 3)))

    src = edge_index[0]
    dst = edge_index[1]
    perm = jnp.argsort(dst)
    sdst = dst[perm]
    ssrc = src[perm]
    srel = edge_type[perm]

    gidx = jnp.pad(ssrc * NREL + srel, (0, E_PAD - E)).astype(jnp.int32)
    code = jnp.pad(sdst * 8 + srel, (0, E_PAD - E),
                   constant_values=8 * N).astype(jnp.int32)
    code3d = code.reshape(NCH, 1, B_EDGE)
    offs = jnp.searchsorted(
        sdst, jnp.arange(NB + 1, dtype=jnp.int32) * R_NODE).astype(jnp.int32)

    d1p, d2p, d3p = 1408, 1024, 768
    w1l = [jnp.pad(w1[r], ((0, 3), (0, 0))) for r in range(NREL)]
    w1p = jnp.stack(w1l)
    root1p = jnp.pad(root1, ((0, 3), (0, 0)))
    h1 = _rgcn_layer(h, w1p, root1p, b1, gidx, code3d, offs, 1616, d1p)
    h2 = _rgcn_layer(h1, w2, root2, b2, gidx, code3d, offs, d1p, d2p)

    lw1p = _pad2(lw1, d2p, d3p)
    lb1p = jnp.pad(lb1, (0, d3p - lb1.shape[0]))[None, :]
    lw2p = _pad2(lw2, d3p, 128)
    lb2p = jnp.pad(lb2, (0, 128 - lb2.shape[0]))[None, :]
    logp_pad, emb_pad = _head(h2, lw1p, lb1p, lw2p, lb2p, d2p, d3p)
    return (logp_pad[:, :2], emb_pad[:, :740])


# speedup vs baseline: 3.7032x; 1.0367x over previous
import functools

import jax
import jax.numpy as jnp
from jax import lax
from jax.experimental import pallas as pl
from jax.experimental.pallas import tpu as pltpu
from jax.experimental.pallas import tpu_sc as plsc

N_X = 5736
N_GENE = 4264
N = N_X + N_GENE
E = 160000
NREL = 5

B_EDGE = 512
R_NODE = 200
NB = N // R_NODE
E_PAD = ((E + B_EDGE - 1) // B_EDGE) * B_EDGE
NCH = E_PAD // B_EDGE


def _mm_body(a_ref, b_ref, o_ref):
    o_ref[...] = jnp.dot(a_ref[...], b_ref[...],
                         preferred_element_type=jnp.float32)


def _matmul(a, b, bm=200, bn=256):
    m, k = a.shape
    k2, n = b.shape
    assert k == k2 and m % bm == 0 and n % bn == 0
    return pl.pallas_call(
        _mm_body,
        grid=(m // bm, n // bn),
        in_specs=[
            pl.BlockSpec((bm, k), lambda i, j: (i, 0)),
            pl.BlockSpec((k, bn), lambda i, j: (0, j)),
        ],
        out_specs=pl.BlockSpec((bm, bn), lambda i, j: (i, j)),
        out_shape=jax.ShapeDtypeStruct((m, n), jnp.float32),
    )(a, b)


def _sc_gather(table, idx, d):
    info = plsc.get_sparse_core_info()
    nc, ns = info.num_cores, info.num_subcores
    nw = nc * ns
    b_per_w = E_PAD // nw
    chunk = 32
    nsteps = b_per_w // chunk
    mesh = plsc.VectorSubcoreMesh(core_axis_name="c", subcore_axis_name="s")

    @functools.partial(
        pl.kernel, mesh=mesh,
        out_type=jax.ShapeDtypeStruct((E_PAD, d), jnp.float32),
        scratch_types=[
            pltpu.VMEM((chunk,), jnp.int32),
            pltpu.VMEM((chunk, d), jnp.float32),
            pltpu.SemaphoreType.DMA,
        ],
    )
    def gather_k(table_hbm, idx_hbm, out_hbm, idx_v, rows_v, sem):
        wid = lax.axis_index("s") * nc + lax.axis_index("c")
        base = wid * b_per_w

        def body(i, carry):
            off = base + i * chunk
            pltpu.sync_copy(idx_hbm.at[pl.ds(off, chunk)], idx_v)
            pltpu.async_copy(table_hbm.at[idx_v], rows_v, sem).wait()
            pltpu.sync_copy(rows_v, out_hbm.at[pl.ds(off, chunk)])
            return carry

        lax.fori_loop(0, nsteps, body, 0)

    return gather_k(table, idx)


def _seg_body(offs_ref, msg_hbm, code_hbm, out0_ref, out_ref,
              code_v, msg_v, sem, *, dp):
    k = pl.program_id(0)
    lo = offs_ref[k]
    hi = offs_ref[k + 1]
    c_lo = lo // B_EDGE
    c_hi = (hi + B_EDGE - 1) // B_EDGE

    iota_col = lax.broadcasted_iota(jnp.int32, (R_NODE, B_EDGE), 0)

    def body(c, carry):
        s, cnt = carry
        pltpu.make_async_copy(code_hbm.at[c], code_v, sem).start()
        pltpu.make_async_copy(code_hbm.at[c], code_v, sem).wait()
        pltpu.make_async_copy(
            msg_hbm.at[(pl.ds(c * B_EDGE, B_EDGE), slice(None))], msg_v,
            sem).start()
        pltpu.make_async_copy(
            msg_hbm.at[(pl.ds(c * B_EDGE, B_EDGE), slice(None))], msg_v,
            sem).wait()
        code = code_v[...]
        dloc = code // 8 - k * R_NODE
        rel = code % 8
        dloc_b = jnp.broadcast_to(dloc, (R_NODE, B_EDGE))
        rel_b = jnp.broadcast_to(rel, (R_NODE, B_EDGE))
        msg = msg_v[...]
        s_new = []
        cnt_new = []
        for r in range(NREL):
            oht = jnp.where((dloc_b == iota_col) & (rel_b == r), 1.0, 0.0)
            s_new.append(s[r] + jnp.dot(oht, msg,
                                        preferred_element_type=jnp.float32))
            cnt_new.append(cnt[r] + jnp.sum(oht, axis=1, keepdims=True))
        return tuple(s_new), tuple(cnt_new)

    zero_s = tuple(jnp.zeros((R_NODE, dp), jnp.float32) for _ in range(NREL))
    zero_c = tuple(jnp.zeros((R_NODE, 1), jnp.float32) for _ in range(NREL))
    s, cnt = lax.fori_loop(c_lo, c_hi, body, (zero_s, zero_c))

    out = out0_ref[...]
    for r in range(NREL):
        out = out + s[r] / jnp.maximum(cnt[r], 1.0)
    out_ref[...] = jnp.maximum(out, 0.0)


def _seg_aggregate(offs, msg, code3d, out0, dp):
    grid_spec = pltpu.PrefetchScalarGridSpec(
        num_scalar_prefetch=1,
        grid=(NB,),
        in_specs=[
            pl.BlockSpec(memory_space=pltpu.MemorySpace.HBM),
            pl.BlockSpec(memory_space=pltpu.MemorySpace.HBM),
            pl.BlockSpec((R_NODE, dp), lambda k, offs: (k, 0)),
        ],
        out_specs=pl.BlockSpec((R_NODE, dp), lambda k, offs: (k, 0)),
        scratch_shapes=[
            pltpu.VMEM((1, B_EDGE), jnp.int32),
            pltpu.VMEM((B_EDGE, dp), jnp.float32),
            pltpu.SemaphoreType.DMA,
        ],
    )
    return pl.pallas_call(
        functools.partial(_seg_body, dp=dp),
        grid_spec=grid_spec,
        out_shape=jax.ShapeDtypeStruct((N, dp), jnp.float32),
    )(offs, msg, code3d, out0)


def _head_body(h_ref, lw1_ref, lb1_ref, lw2_ref, lb2_ref, logp_ref, emb_ref):
    z1 = jnp.maximum(
        jnp.dot(h_ref[...], lw1_ref[...], preferred_element_type=jnp.float32)
        + lb1_ref[...], 0.0)
    logits = jnp.dot(z1, lw2_ref[...],
                     preferred_element_type=jnp.float32) + lb2_ref[...]
    l0 = logits[:, 0:1]
    l1 = logits[:, 1:2]
    m = jnp.maximum(l0, l1)
    lse = m + jnp.log(jnp.exp(l0 - m) + jnp.exp(l1 - m))
    logp_ref[...] = logits - lse
    emb_ref[...] = z1


def _head(h, lw1p, lb1p, lw2p, lb2p, dp, d3p):
    return pl.pallas_call(
        _head_body,
        grid=(NB,),
        in_specs=[
            pl.BlockSpec((R_NODE, dp), lambda k: (k, 0)),
            pl.BlockSpec((dp, d3p), lambda k: (0, 0)),
            pl.BlockSpec((1, d3p), lambda k: (0, 0)),
            pl.BlockSpec((d3p, 128), lambda k: (0, 0)),
            pl.BlockSpec((1, 128), lambda k: (0, 0)),
        ],
        out_specs=(
            pl.BlockSpec((R_NODE, 128), lambda k: (k, 0)),
            pl.BlockSpec((R_NODE, d3p), lambda k: (k, 0)),
        ),
        out_shape=(
            jax.ShapeDtypeStruct((N, 128), jnp.float32),
            jax.ShapeDtypeStruct((N, d3p), jnp.float32),
        ),
    )(h, lw1p, lb1p, lw2p, lb2p)


def _pad2(a, rows, cols):
    return jnp.pad(a, ((0, rows - a.shape[0]), (0, cols - a.shape[1])))


def _rgcn_layer(h, w, root, b, gidx, code3d, offs, din_p, dout_p):
    blocks = [_pad2(w[r], din_p, dout_p) for r in range(NREL)]
    blocks.append(_pad2(root, din_p, dout_p))
    wcat = jnp.concatenate(blocks, axis=1)
    t = _matmul(h, wcat)
    table = t[:, :NREL * dout_p].reshape(N * NREL, dout_p)
    bp = jnp.pad(b, (0, dout_p - b.shape[0]))
    out0 = t[:, NREL * dout_p:] + bp[None, :]
    msg = _sc_gather(table, gidx, dout_p)
    return _seg_aggregate(offs, msg, code3d, out0, dout_p)


def kernel(x, edge_index, edge_type, gene_emb, w1, root1, b1, w2, root2, b2,
           lw1, lb1, lw2, lb2):
    h = jnp.concatenate([x, gene_emb], axis=0)
    h = jnp.pad(h, ((0, 0), (0, 3)))

    src = edge_index[0]
    dst = edge_index[1]
    perm = jnp.argsort(dst)
    sdst = dst[perm]
    ssrc = src[perm]
    srel = edge_type[perm]

    gidx = jnp.pad(ssrc * NREL + srel, (0, E_PAD - E)).astype(jnp.int32)
    code = jnp.pad(sdst * 8 + srel, (0, E_PAD - E),
                   constant_values=8 * N).astype(jnp.int32)
    code3d = code.reshape(NCH, 1, B_EDGE)
    offs = jnp.searchsorted(
        sdst, jnp.arange(NB + 1, dtype=jnp.int32) * R_NODE).astype(jnp.int32)

    d1p, d2p, d3p = 1408, 1024, 768
    w1l = [jnp.pad(w1[r], ((0, 3), (0, 0))) for r in range(NREL)]
    w1p = jnp.stack(w1l)
    root1p = jnp.pad(root1, ((0, 3), (0, 0)))
    h1 = _rgcn_layer(h, w1p, root1p, b1, gidx, code3d, offs, 1616, d1p)
    h2 = _rgcn_layer(h1, w2, root2, b2, gidx, code3d, offs, d1p, d2p)

    lw1p = _pad2(lw1, d2p, d3p)
    lb1p = jnp.pad(lb1, (0, d3p - lb1.shape[0]))[None, :]
    lw2p = _pad2(lw2, d3p, 128)
    lb2p = jnp.pad(lb2, (0, 128 - lb2.shape[0]))[None, :]
    logp_pad, emb_pad = _head(h2, lw1p, lb1p, lw2p, lb2p, d2p, d3p)
    return (logp_pad[:, :2], emb_pad[:, :740])
